# baseline (speedup 1.0000x reference)
import functools

import jax
import jax.numpy as jnp
from jax import lax
from jax.experimental import pallas as pl
from jax.experimental.pallas import tpu as pltpu
from jax.experimental.pallas import tpu_sc as plsc

OUT_F = 4096
IN_F = 4096

NUM_CORES = 2
NUM_TILES = 16
LANES = 16
SCAT_B = 128

CHUNKS = (2560, 1536)
ROW_BLK = 256
BLK_WORDS = ROW_BLK * IN_F
TILE_SLICE = BLK_WORDS // NUM_TILES
ROWS_PT = TILE_SLICE // IN_F
DUMMY_PAD = NUM_TILES * SCAT_B
ZBUF = 16384


def _densify(r3, c3, w3, nb, row_start, chunk_rows):
    passes = chunk_rows // ROW_BLK // NUM_CORES
    mesh = plsc.VectorSubcoreMesh(
        core_axis_name="c", subcore_axis_name="s",
        num_cores=NUM_CORES, num_subcores=NUM_TILES)

    @functools.partial(
        pl.kernel,
        out_type=jax.ShapeDtypeStruct((chunk_rows, IN_F), jnp.float32),
        mesh=mesh,
        scratch_types=[
            pltpu.VMEM((nb, SCAT_B), jnp.int32),
            pltpu.VMEM((nb, SCAT_B), jnp.int32),
            pltpu.VMEM((nb, SCAT_B), jnp.float32),
            pltpu.VMEM((nb, SCAT_B), jnp.int32),
            pltpu.VMEM((ZBUF,), jnp.float32),
            pltpu.VMEM_SHARED((BLK_WORDS + DUMMY_PAD,), jnp.float32),
            pltpu.SemaphoreType.DMA,
            pltpu.SemaphoreType.DMA,
        ],
    )
    def densify_kernel(r_hbm, c_hbm, w_hbm, w_out, r_v, c_v, w_v, idx_v, z_v,
                       shared, sem, sem2):
        cid = lax.axis_index("c")
        sid = lax.axis_index("s")
        pltpu.sync_copy(r_hbm.at[sid], r_v)
        pltpu.sync_copy(c_hbm.at[sid], c_v)
        pltpu.sync_copy(w_hbm.at[sid], w_v)

        def _zero(i, _):
            z_v[pl.ds(i * LANES, LANES)] = jnp.zeros((LANES,), jnp.float32)
            return 0
        lax.fori_loop(0, ZBUF // LANES, _zero, 0)

        iota = lax.iota(jnp.int32, LANES)
        dums = [BLK_WORDS + sid * SCAT_B + k * LANES + iota
                for k in range(SCAT_B // LANES)]

        def _flat(j, _):
            def _flat2(k, _2):
                sl = pl.ds(k * LANES, LANES)
                r_v[j, sl] = r_v[j, sl] * IN_F + c_v[j, sl]
                return 0
            return lax.fori_loop(0, SCAT_B // LANES, _flat2, 0)
        lax.fori_loop(0, nb, _flat, 0)

        for p in range(passes):
            g = cid * passes + p
            base = (row_start + g * ROW_BLK) * IN_F
            zcps = [
                pltpu.async_copy(
                    z_v,
                    shared.at[pl.ds(sid * TILE_SLICE + z * ZBUF, ZBUF)],
                    sem2)
                for z in range(TILE_SLICE // ZBUF)
            ]
            for cp in zcps:
                cp.wait()
            plsc.subcore_barrier()

            def _cidx(j, _):
                for k in range(SCAT_B // LANES):
                    sl = pl.ds(k * LANES, LANES)
                    off = r_v[j, sl] - base
                    inb = (off >= 0) & (off < BLK_WORDS)
                    idx_v[j, sl] = jnp.where(inb, off, dums[k])
                pltpu.async_copy(w_v.at[j], shared.at[idx_v.at[j]], sem2,
                                 add=True)
                return 0
            lax.fori_loop(0, nb, _cidx, 0)

            def _drain(j, _):
                pltpu.make_async_copy(
                    w_v.at[0], shared.at[idx_v.at[0]], sem2).wait()
                return 0
            lax.fori_loop(0, nb, _drain, 0)
            plsc.subcore_barrier()

            row_base = g * ROW_BLK + sid * ROWS_PT
            copies = [
                pltpu.async_copy(
                    shared.at[pl.ds((sid * ROWS_PT + rr) * IN_F, IN_F)],
                    w_out.at[row_base + rr], sem)
                for rr in range(ROWS_PT)
            ]
            for cp in copies:
                cp.wait()

    return densify_kernel(r3, c3, w3)


BN = 512


def _mm_body(x_ref, w_ref, b_ref, o_ref):
    o_ref[...] = lax.dot_general(
        x_ref[...].astype(jnp.bfloat16), w_ref[...].astype(jnp.bfloat16),
        (((1,), (1,)), ((), ())),
        preferred_element_type=jnp.float32) + b_ref[...]


def _mm_body_alias(prev_ref, x_ref, w_ref, b_ref, o_ref):
    del prev_ref
    _mm_body(x_ref, w_ref, b_ref, o_ref)


def _matmul(prev, x, w, b2, col_blk0):
    m = x.shape[0]
    n = w.shape[0]
    specs = [
        pl.BlockSpec((m, IN_F), lambda j: (0, 0)),
        pl.BlockSpec((BN, IN_F), lambda j: (j, 0)),
        pl.BlockSpec((1, BN), lambda j: (0, j + col_blk0)),
    ]
    if prev is None:
        body, args, specs, aliases = _mm_body, (x, w, b2), specs, {}
    else:
        body = _mm_body_alias
        args = (prev, x, w, b2)
        specs = [pl.BlockSpec(memory_space=pl.ANY)] + specs
        aliases = {0: 0}
    return pl.pallas_call(
        body,
        grid=(n // BN,),
        in_specs=specs,
        out_specs=pl.BlockSpec((m, BN), lambda j: (0, j + col_blk0)),
        out_shape=jax.ShapeDtypeStruct((m, OUT_F), jnp.float32),
        input_output_aliases=aliases,
        compiler_params=pltpu.CompilerParams(
            dimension_semantics=("arbitrary",)),
    )(*args)


def kernel(inputs, weights, bias, rows, cols):
    nnz = rows.shape[0]
    per_tile = -(-nnz // (NUM_TILES * SCAT_B)) * SCAT_B
    nb = per_tile // SCAT_B
    pad = NUM_TILES * per_tile - nnz

    r = jnp.pad(rows.astype(jnp.int32), (0, pad))
    c = jnp.pad(cols.astype(jnp.int32), (0, pad))
    w = jnp.pad(weights.astype(jnp.float32), (0, pad))

    r3 = r.reshape(NUM_TILES, nb, SCAT_B)
    c3 = c.reshape(NUM_TILES, nb, SCAT_B)
    w3 = w.reshape(NUM_TILES, nb, SCAT_B)

    x = inputs.reshape(-1, IN_F)
    m = x.shape[0]
    b2 = bias.reshape(1, OUT_F)
    out = None
    row0 = 0
    for chunk_rows in CHUNKS:
        w_chunk = _densify(r3, c3, w3, nb, row0, chunk_rows)
        out = _matmul(out, x, w_chunk, b2, row0 // BN)
        row0 += chunk_rows
    return out.reshape(*inputs.shape[:-1], OUT_F)


# speedup vs baseline: 1.0065x; 1.0065x over previous
import functools

import jax
import jax.numpy as jnp
from jax import lax
from jax.experimental import pallas as pl
from jax.experimental.pallas import tpu as pltpu
from jax.experimental.pallas import tpu_sc as plsc

OUT_F = 4096
IN_F = 4096

NUM_CORES = 2
NUM_TILES = 16
LANES = 16
SCAT_B = 128

CHUNKS = (3072, 1024)
ROW_BLK = 256
BLK_WORDS = ROW_BLK * IN_F
TILE_SLICE = BLK_WORDS // NUM_TILES
ROWS_PT = TILE_SLICE // IN_F
DUMMY_PAD = NUM_TILES * SCAT_B
ZBUF = 16384


def _densify(r3, c3, w3, nb, row_start, chunk_rows):
    passes = chunk_rows // ROW_BLK // NUM_CORES
    mesh = plsc.VectorSubcoreMesh(
        core_axis_name="c", subcore_axis_name="s",
        num_cores=NUM_CORES, num_subcores=NUM_TILES)

    @functools.partial(
        pl.kernel,
        out_type=jax.ShapeDtypeStruct((chunk_rows, IN_F), jnp.float32),
        mesh=mesh,
        scratch_types=[
            pltpu.VMEM((nb, SCAT_B), jnp.int32),
            pltpu.VMEM((nb, SCAT_B), jnp.int32),
            pltpu.VMEM((nb, SCAT_B), jnp.float32),
            pltpu.VMEM((nb, SCAT_B), jnp.int32),
            pltpu.VMEM((ZBUF,), jnp.float32),
            pltpu.VMEM_SHARED((BLK_WORDS + DUMMY_PAD,), jnp.float32),
            pltpu.SemaphoreType.DMA,
            pltpu.SemaphoreType.DMA,
        ],
    )
    def densify_kernel(r_hbm, c_hbm, w_hbm, w_out, r_v, c_v, w_v, idx_v, z_v,
                       shared, sem, sem2):
        cid = lax.axis_index("c")
        sid = lax.axis_index("s")
        pltpu.sync_copy(r_hbm.at[sid], r_v)
        pltpu.sync_copy(c_hbm.at[sid], c_v)
        pltpu.sync_copy(w_hbm.at[sid], w_v)

        def _zero(i, _):
            z_v[pl.ds(i * LANES, LANES)] = jnp.zeros((LANES,), jnp.float32)
            return 0
        lax.fori_loop(0, ZBUF // LANES, _zero, 0)

        iota = lax.iota(jnp.int32, LANES)
        dums = [BLK_WORDS + sid * SCAT_B + k * LANES + iota
                for k in range(SCAT_B // LANES)]

        def _flat(j, _):
            def _flat2(k, _2):
                sl = pl.ds(k * LANES, LANES)
                r_v[j, sl] = r_v[j, sl] * IN_F + c_v[j, sl]
                return 0
            return lax.fori_loop(0, SCAT_B // LANES, _flat2, 0)
        lax.fori_loop(0, nb, _flat, 0)

        for p in range(passes):
            g = cid * passes + p
            base = (row_start + g * ROW_BLK) * IN_F
            zcps = [
                pltpu.async_copy(
                    z_v,
                    shared.at[pl.ds(sid * TILE_SLICE + z * ZBUF, ZBUF)],
                    sem2)
                for z in range(TILE_SLICE // ZBUF)
            ]
            for cp in zcps:
                cp.wait()
            plsc.subcore_barrier()

            def _cidx(j, _):
                for k in range(SCAT_B // LANES):
                    sl = pl.ds(k * LANES, LANES)
                    off = r_v[j, sl] - base
                    inb = (off >= 0) & (off < BLK_WORDS)
                    idx_v[j, sl] = jnp.where(inb, off, dums[k])
                pltpu.async_copy(w_v.at[j], shared.at[idx_v.at[j]], sem2,
                                 add=True)
                return 0
            lax.fori_loop(0, nb, _cidx, 0)

            def _drain(j, _):
                pltpu.make_async_copy(
                    w_v.at[0], shared.at[idx_v.at[0]], sem2).wait()
                return 0
            lax.fori_loop(0, nb, _drain, 0)
            plsc.subcore_barrier()

            row_base = g * ROW_BLK + sid * ROWS_PT
            copies = [
                pltpu.async_copy(
                    shared.at[pl.ds((sid * ROWS_PT + rr) * IN_F, IN_F)],
                    w_out.at[row_base + rr], sem)
                for rr in range(ROWS_PT)
            ]
            for cp in copies:
                cp.wait()

    return densify_kernel(r3, c3, w3)


BN = 512


def _mm_body(x_ref, w_ref, b_ref, o_ref):
    o_ref[...] = lax.dot_general(
        x_ref[...].astype(jnp.bfloat16), w_ref[...].astype(jnp.bfloat16),
        (((1,), (1,)), ((), ())),
        preferred_element_type=jnp.float32) + b_ref[...]


def _mm_body_alias(prev_ref, x_ref, w_ref, b_ref, o_ref):
    del prev_ref
    _mm_body(x_ref, w_ref, b_ref, o_ref)


def _matmul(prev, x, w, b2, col_blk0):
    m = x.shape[0]
    n = w.shape[0]
    specs = [
        pl.BlockSpec((m, IN_F), lambda j: (0, 0)),
        pl.BlockSpec((BN, IN_F), lambda j: (j, 0)),
        pl.BlockSpec((1, BN), lambda j: (0, j + col_blk0)),
    ]
    if prev is None:
        body, args, specs, aliases = _mm_body, (x, w, b2), specs, {}
    else:
        body = _mm_body_alias
        args = (prev, x, w, b2)
        specs = [pl.BlockSpec(memory_space=pl.ANY)] + specs
        aliases = {0: 0}
    return pl.pallas_call(
        body,
        grid=(n // BN,),
        in_specs=specs,
        out_specs=pl.BlockSpec((m, BN), lambda j: (0, j + col_blk0)),
        out_shape=jax.ShapeDtypeStruct((m, OUT_F), jnp.float32),
        input_output_aliases=aliases,
        compiler_params=pltpu.CompilerParams(
            dimension_semantics=("arbitrary",)),
    )(*args)


def kernel(inputs, weights, bias, rows, cols):
    nnz = rows.shape[0]
    per_tile = -(-nnz // (NUM_TILES * SCAT_B)) * SCAT_B
    nb = per_tile // SCAT_B
    pad = NUM_TILES * per_tile - nnz

    r = jnp.pad(rows.astype(jnp.int32), (0, pad))
    c = jnp.pad(cols.astype(jnp.int32), (0, pad))
    w = jnp.pad(weights.astype(jnp.float32), (0, pad))

    r3 = r.reshape(NUM_TILES, nb, SCAT_B)
    c3 = c.reshape(NUM_TILES, nb, SCAT_B)
    w3 = w.reshape(NUM_TILES, nb, SCAT_B)

    x = inputs.reshape(-1, IN_F)
    m = x.shape[0]
    b2 = bias.reshape(1, OUT_F)
    out = None
    row0 = 0
    for chunk_rows in CHUNKS:
        w_chunk = _densify(r3, c3, w3, nb, row0, chunk_rows)
        out = _matmul(out, x, w_chunk, b2, row0 // BN)
        row0 += chunk_rows
    return out.reshape(*inputs.shape[:-1], OUT_F)
